# baseline (speedup 1.0000x reference)
import functools

import jax
import jax.numpy as jnp
from jax import lax
from jax.experimental import pallas as pl
from jax.experimental.pallas import tpu as pltpu
from jax.experimental.pallas import tpu_sc as plsc

_BEAM = 4
_VOCAB = 100000
_PAD_V = 100096
_NEG = -1e30
_HEADS = 16
_SEQ = 2048
_HDIM = 64
_NKV = 12
_SC_CHUNK = 512


def _topk_rp_body(logits_ref, rp_ref, pen_ref, idx_ref, prob_ref, rp_out_ref):
    x = logits_ref[...]
    m = jnp.max(x)
    lse = jnp.log(jnp.sum(jnp.exp(x - m))) + m
    cols = lax.broadcasted_iota(jnp.int32, (1, _PAD_V), 1)
    vals = []
    idxs = []
    xc = x
    for _ in range(_BEAM):
        mk = jnp.max(xc)
        ik = jnp.min(jnp.where(xc == mk, cols, _PAD_V))
        vals.append(mk)
        idxs.append(ik)
        xc = jnp.where(cols == ik, _NEG, xc)
    for k in range(_BEAM):
        idx_ref[k, 0] = idxs[k]
        prob_ref[k, 0] = vals[k] - lse
    rcols = lax.broadcasted_iota(jnp.int32, (1, _VOCAB), 1)
    mask = (
        (rcols == idxs[0]) | (rcols == idxs[1]) | (rcols == idxs[2]) | (rcols == idxs[3])
    )
    p = pen_ref[0]
    rp_out_ref[...] = rp_ref[...] * jnp.where(mask, p, jnp.float32(1.0))


def _topk_rp(logits, rp, pen):
    logits_pad = jnp.pad(logits, ((0, 0), (0, _PAD_V - _VOCAB)), constant_values=_NEG)
    return pl.pallas_call(
        _topk_rp_body,
        in_specs=[
            pl.BlockSpec(memory_space=pltpu.MemorySpace.VMEM),
            pl.BlockSpec(memory_space=pltpu.MemorySpace.VMEM),
            pl.BlockSpec(memory_space=pltpu.MemorySpace.SMEM),
        ],
        out_specs=[
            pl.BlockSpec(memory_space=pltpu.MemorySpace.SMEM),
            pl.BlockSpec(memory_space=pltpu.MemorySpace.SMEM),
            pl.BlockSpec(memory_space=pltpu.MemorySpace.VMEM),
        ],
        out_shape=[
            jax.ShapeDtypeStruct((_BEAM, 1), jnp.int32),
            jax.ShapeDtypeStruct((_BEAM, 1), jnp.float32),
            jax.ShapeDtypeStruct((_BEAM, _VOCAB), jnp.float32),
        ],
    )(logits_pad, rp, pen)


def _sc_bcast_body(*refs):
    kv_in = refs[:_NKV]
    kv_out = refs[_NKV:2 * _NKV]
    buf, rsem, wsem = refs[2 * _NKV:]

    cid_core = lax.axis_index("c")
    sid = lax.axis_index("s")
    wid = sid * 2 + cid_core

    nchunk = _SEQ // _SC_CHUNK

    def src_dst(t):
        i, r = divmod(t, 2)
        cid = wid * 2 + r
        h = lax.shift_right_logical(cid, 2)
        s0 = (cid & (nchunk - 1)) * _SC_CHUNK
        src = kv_in[i].at[0, h, pl.ds(s0, _SC_CHUNK)]
        dsts = [kv_out[i].at[b, h, pl.ds(s0, _SC_CHUNK)] for b in range(_BEAM)]
        return src, dsts

    total = _NKV * 2
    reads = {}
    pend = {}

    src0, _ = src_dst(0)
    reads[0] = pltpu.make_async_copy(src0, buf.at[0], rsem.at[0])
    reads[0].start()

    for t in range(total):
        slot = t % 2
        nslot = (t + 1) % 2
        if t >= 1:
            for w in pend[t - 1]:
                w.wait()
        if t + 1 < total:
            srcn, _ = src_dst(t + 1)
            reads[t + 1] = pltpu.make_async_copy(srcn, buf.at[nslot], rsem.at[nslot])
            reads[t + 1].start()
        reads[t].wait()
        _, dsts = src_dst(t)
        ws = [pltpu.make_async_copy(buf.at[slot], d, wsem.at[slot]) for d in dsts]
        for w in ws:
            w.start()
        pend[t] = ws
    for w in pend[total - 1]:
        w.wait()


def _sc_bcast(kvs):
    f = pl.kernel(
        _sc_bcast_body,
        out_type=[
            jax.ShapeDtypeStruct((_BEAM, _HEADS, _SEQ, _HDIM), jnp.float32)
        ] * _NKV,
        mesh=plsc.VectorSubcoreMesh(core_axis_name="c", subcore_axis_name="s"),
        scratch_types=[
            pltpu.VMEM((2, _SC_CHUNK, _HDIM), jnp.float32),
            pltpu.SemaphoreType.DMA((2,)),
            pltpu.SemaphoreType.DMA((2,)),
        ],
    )
    return f(*kvs)


def kernel(kv_0, kv_1, kv_2, kv_3, kv_4, kv_5, kv_6, kv_7, kv_8, kv_9, kv_10,
           kv_11, logits, save_id, repeat_penality, penality_value, beam_size):
    kvs = [kv_0, kv_1, kv_2, kv_3, kv_4, kv_5, kv_6, kv_7, kv_8, kv_9, kv_10, kv_11]
    tiled = _sc_bcast(kvs)
    idx, prob, rp_out = _topk_rp(logits, repeat_penality, penality_value)
    save_id_out = jnp.concatenate([save_id, idx], axis=-1)
    batch_indices = jnp.arange(_BEAM, dtype=jnp.int32) + (beam_size - _BEAM)
    max_logits_idx = idx[0]
    return (*tiled, idx, save_id_out, rp_out, prob, batch_indices, max_logits_idx)


# speedup vs baseline: 5.3381x; 5.3381x over previous
import functools

import jax
import jax.numpy as jnp
from jax import lax
from jax.experimental import pallas as pl
from jax.experimental.pallas import tpu as pltpu
from jax.experimental.pallas import tpu_sc as plsc

_BEAM = 4
_VOCAB = 100000
_PAD_V = 100096
_NEG = -1e30
_HEADS = 16
_SEQ = 2048
_HDIM = 64
_NKV = 12
_SC_ROWS = 16


def _topk_rp_body(logits_ref, rp_ref, pen_ref, idx_ref, prob_ref, rp_out_ref):
    x = logits_ref[...]
    m = jnp.max(x)
    lse = jnp.log(jnp.sum(jnp.exp(x - m))) + m
    cols = lax.broadcasted_iota(jnp.int32, (1, _PAD_V), 1)
    vals = []
    idxs = []
    xc = x
    for _ in range(_BEAM):
        mk = jnp.max(xc)
        ik = jnp.min(jnp.where(xc == mk, cols, _PAD_V))
        vals.append(mk)
        idxs.append(ik)
        xc = jnp.where(cols == ik, _NEG, xc)
    for k in range(_BEAM):
        idx_ref[k, 0] = idxs[k]
        prob_ref[k, 0] = vals[k] - lse
    rcols = lax.broadcasted_iota(jnp.int32, (1, _VOCAB), 1)
    mask = (
        (rcols == idxs[0]) | (rcols == idxs[1]) | (rcols == idxs[2]) | (rcols == idxs[3])
    )
    p = pen_ref[0]
    rp_out_ref[...] = rp_ref[...] * jnp.where(mask, p, jnp.float32(1.0))


def _topk_rp(logits, rp, pen):
    logits_pad = jnp.pad(logits, ((0, 0), (0, _PAD_V - _VOCAB)), constant_values=_NEG)
    return pl.pallas_call(
        _topk_rp_body,
        in_specs=[
            pl.BlockSpec(memory_space=pltpu.MemorySpace.VMEM),
            pl.BlockSpec(memory_space=pltpu.MemorySpace.VMEM),
            pl.BlockSpec(memory_space=pltpu.MemorySpace.SMEM),
        ],
        out_specs=[
            pl.BlockSpec(memory_space=pltpu.MemorySpace.SMEM),
            pl.BlockSpec(memory_space=pltpu.MemorySpace.SMEM),
            pl.BlockSpec(memory_space=pltpu.MemorySpace.VMEM),
        ],
        out_shape=[
            jax.ShapeDtypeStruct((_BEAM, 1), jnp.int32),
            jax.ShapeDtypeStruct((_BEAM, 1), jnp.float32),
            jax.ShapeDtypeStruct((_BEAM, _VOCAB), jnp.float32),
        ],
    )(logits_pad, rp, pen)


def _sc_bcast_body(*refs):
    kv_in = refs[:_NKV]
    kv_out = refs[_NKV:2 * _NKV]
    buf, rsem, wsem = refs[2 * _NKV:]

    cid_core = lax.axis_index("c")
    sid = lax.axis_index("s")
    wid = sid * 2 + cid_core

    nchunk = _HDIM // _SC_ROWS

    def src_dst(t):
        i, r = divmod(t, 2)
        cid = wid * 2 + r
        h = lax.shift_right_logical(cid, 2)
        r0 = (cid & (nchunk - 1)) * _SC_ROWS
        src = kv_in[i].at[0, h, pl.ds(r0, _SC_ROWS)]
        dsts = [kv_out[i].at[b, h, pl.ds(r0, _SC_ROWS)] for b in range(_BEAM)]
        return src, dsts

    total = _NKV * 2
    reads = {}
    pend = {}

    src0, _ = src_dst(0)
    reads[0] = pltpu.make_async_copy(src0, buf.at[0], rsem.at[0])
    reads[0].start()

    for t in range(total):
        slot = t % 2
        nslot = (t + 1) % 2
        if t >= 1:
            for w in pend[t - 1]:
                w.wait()
        if t + 1 < total:
            srcn, _ = src_dst(t + 1)
            reads[t + 1] = pltpu.make_async_copy(srcn, buf.at[nslot], rsem.at[nslot])
            reads[t + 1].start()
        reads[t].wait()
        _, dsts = src_dst(t)
        ws = [pltpu.make_async_copy(buf.at[slot], d, wsem.at[slot]) for d in dsts]
        for w in ws:
            w.start()
        pend[t] = ws
    for w in pend[total - 1]:
        w.wait()


def _sc_bcast(kvs):
    f = pl.kernel(
        _sc_bcast_body,
        out_type=[
            jax.ShapeDtypeStruct((_BEAM, _HEADS, _HDIM, _SEQ), jnp.float32)
        ] * _NKV,
        mesh=plsc.VectorSubcoreMesh(core_axis_name="c", subcore_axis_name="s"),
        scratch_types=[
            pltpu.VMEM((2, _SC_ROWS, _SEQ), jnp.float32),
            pltpu.SemaphoreType.DMA((2,)),
            pltpu.SemaphoreType.DMA((2,)),
        ],
    )
    return f(*kvs)


def kernel(kv_0, kv_1, kv_2, kv_3, kv_4, kv_5, kv_6, kv_7, kv_8, kv_9, kv_10,
           kv_11, logits, save_id, repeat_penality, penality_value, beam_size):
    kvs = [kv_0, kv_1, kv_2, kv_3, kv_4, kv_5, kv_6, kv_7, kv_8, kv_9, kv_10, kv_11]
    kvts = [jnp.swapaxes(kv, 2, 3) for kv in kvs]
    tiled_t = _sc_bcast(kvts)
    tiled = [jnp.swapaxes(o, 2, 3) for o in tiled_t]
    idx, prob, rp_out = _topk_rp(logits, repeat_penality, penality_value)
    save_id_out = jnp.concatenate([save_id, idx], axis=-1)
    batch_indices = jnp.arange(_BEAM, dtype=jnp.int32) + (beam_size - _BEAM)
    max_logits_idx = idx[0]
    return (*tiled, idx, save_id_out, rp_out, prob, batch_indices, max_logits_idx)


# speedup vs baseline: 5.4981x; 1.0300x over previous
import jax
import jax.numpy as jnp
from jax import lax
from jax.experimental import pallas as pl
from jax.experimental.pallas import tpu as pltpu
from jax.experimental.pallas import tpu_sc as plsc

_BEAM = 4
_VOCAB = 100000
_PAD_V = 100096
_NEG = -1e30
_HEADS = 16
_SEQ = 2048
_HDIM = 64
_NKV = 12
_TC_N = 2
_SC_N = _NKV - _TC_N
_SC_ROWS = 16
_TC_CH = 4
_TC_NCH = _HEADS // _TC_CH


def _tc_body(*refs):
    kv_in = refs[:_TC_N]
    logits_ref, rp_ref, pen_ref = refs[_TC_N:_TC_N + 3]
    kv_out = refs[_TC_N + 3:2 * _TC_N + 3]
    idx_ref, prob_ref, rp_out_ref = refs[2 * _TC_N + 3:2 * _TC_N + 6]
    bufs = refs[2 * _TC_N + 6]
    rsem = refs[2 * _TC_N + 7]
    wsems = refs[2 * _TC_N + 8:2 * _TC_N + 12]

    total = _TC_N * _TC_NCH

    def read_for(t, slot):
        i, c = divmod(t, _TC_NCH)
        return pltpu.make_async_copy(
            kv_in[i].at[0, pl.ds(c * _TC_CH, _TC_CH)], bufs.at[slot], rsem.at[slot])

    def writes_for(t, slot):
        i, c = divmod(t, _TC_NCH)
        return [
            pltpu.make_async_copy(
                bufs.at[slot], kv_out[i].at[b, pl.ds(c * _TC_CH, _TC_CH)],
                wsems[b].at[slot])
            for b in range(_BEAM)
        ]

    rd = {0: read_for(0, 0), 1: read_for(1, 1)}
    rd[0].start()
    rd[1].start()

    x = logits_ref[...]
    m = jnp.max(x)
    lse = jnp.log(jnp.sum(jnp.exp(x - m))) + m
    cols = lax.broadcasted_iota(jnp.int32, (1, _PAD_V), 1)
    vals, idxs = [], []
    xc = x
    for _ in range(_BEAM):
        mk = jnp.max(xc)
        ik = jnp.min(jnp.where(xc == mk, cols, _PAD_V))
        vals.append(mk)
        idxs.append(ik)
        xc = jnp.where(cols == ik, _NEG, xc)
    for k in range(_BEAM):
        idx_ref[k, 0] = idxs[k]
        prob_ref[k, 0] = vals[k] - lse
    rcols = lax.broadcasted_iota(jnp.int32, (1, _VOCAB), 1)
    mask = (
        (rcols == idxs[0]) | (rcols == idxs[1]) | (rcols == idxs[2]) | (rcols == idxs[3])
    )
    p = pen_ref[0]
    rp_out_ref[...] = rp_ref[...] * jnp.where(mask, p, jnp.float32(1.0))

    pending = {}
    for t in range(total):
        slot = t % 2
        if t >= 2:
            for w in pending[t - 2]:
                w.wait()
            rd[t] = read_for(t, slot)
            rd[t].start()
        rd[t].wait()
        ws = writes_for(t, slot)
        for w in ws:
            w.start()
        pending[t] = ws
    for t in (total - 2, total - 1):
        for w in pending[t]:
            w.wait()


def _tc_call(kvs_tc, logits, rp, pen):
    logits_pad = jnp.pad(logits, ((0, 0), (0, _PAD_V - _VOCAB)), constant_values=_NEG)
    hbm = pl.BlockSpec(memory_space=pltpu.MemorySpace.HBM)
    vmem = pl.BlockSpec(memory_space=pltpu.MemorySpace.VMEM)
    smem = pl.BlockSpec(memory_space=pltpu.MemorySpace.SMEM)
    return pl.pallas_call(
        _tc_body,
        in_specs=[hbm] * _TC_N + [vmem, vmem, smem],
        out_specs=[hbm] * _TC_N + [smem, smem, vmem],
        out_shape=(
            [jax.ShapeDtypeStruct((_BEAM, _HEADS, _HDIM, _SEQ), jnp.float32)] * _TC_N
            + [
                jax.ShapeDtypeStruct((_BEAM, 1), jnp.int32),
                jax.ShapeDtypeStruct((_BEAM, 1), jnp.float32),
                jax.ShapeDtypeStruct((_BEAM, _VOCAB), jnp.float32),
            ]
        ),
        scratch_shapes=[
            pltpu.VMEM((2, _TC_CH, _HDIM, _SEQ), jnp.float32),
            pltpu.SemaphoreType.DMA((2,)),
        ] + [pltpu.SemaphoreType.DMA((2,))] * _BEAM,
    )(*kvs_tc, logits_pad, rp, pen)


def _sc_body(*refs):
    kv_in = refs[:_SC_N]
    kv_out = refs[_SC_N:2 * _SC_N]
    buf, rsem, wsem = refs[2 * _SC_N:]

    cid_core = lax.axis_index("c")
    sid = lax.axis_index("s")
    wid = sid * 2 + cid_core
    nchunk = _HDIM // _SC_ROWS

    def src_dst(t):
        i, r = divmod(t, 2)
        cid = wid * 2 + r
        h = lax.shift_right_logical(cid, 2)
        r0 = (cid & (nchunk - 1)) * _SC_ROWS
        src = kv_in[i].at[0, h, pl.ds(r0, _SC_ROWS)]
        dsts = [kv_out[i].at[b, h, pl.ds(r0, _SC_ROWS)] for b in range(_BEAM)]
        return src, dsts

    total = _SC_N * 2
    reads = {}
    pend = {}
    src0, _ = src_dst(0)
    reads[0] = pltpu.make_async_copy(src0, buf.at[0], rsem.at[0])
    reads[0].start()
    for t in range(total):
        slot = t % 2
        nslot = (t + 1) % 2
        if t >= 1:
            for w in pend[t - 1]:
                w.wait()
        if t + 1 < total:
            srcn, _ = src_dst(t + 1)
            reads[t + 1] = pltpu.make_async_copy(srcn, buf.at[nslot], rsem.at[nslot])
            reads[t + 1].start()
        reads[t].wait()
        _, dsts = src_dst(t)
        ws = [pltpu.make_async_copy(buf.at[slot], d, wsem.at[slot]) for d in dsts]
        for w in ws:
            w.start()
        pend[t] = ws
    for w in pend[total - 1]:
        w.wait()


def _sc_call(kvs_sc):
    f = pl.kernel(
        _sc_body,
        out_type=[
            jax.ShapeDtypeStruct((_BEAM, _HEADS, _HDIM, _SEQ), jnp.float32)
        ] * _SC_N,
        mesh=plsc.VectorSubcoreMesh(core_axis_name="c", subcore_axis_name="s"),
        scratch_types=[
            pltpu.VMEM((2, _SC_ROWS, _SEQ), jnp.float32),
            pltpu.SemaphoreType.DMA((2,)),
            pltpu.SemaphoreType.DMA((2,)),
        ],
    )
    return f(*kvs_sc)


def kernel(kv_0, kv_1, kv_2, kv_3, kv_4, kv_5, kv_6, kv_7, kv_8, kv_9, kv_10,
           kv_11, logits, save_id, repeat_penality, penality_value, beam_size):
    kvs = [kv_0, kv_1, kv_2, kv_3, kv_4, kv_5, kv_6, kv_7, kv_8, kv_9, kv_10, kv_11]
    kvts = [jnp.swapaxes(kv, 2, 3) for kv in kvs]
    sc_out = _sc_call(kvts[_TC_N:])
    tc_out = _tc_call(kvts[:_TC_N], logits, repeat_penality, penality_value)
    tiled = [jnp.swapaxes(o, 2, 3) for o in list(tc_out[:_TC_N]) + list(sc_out)]
    idx, prob, rp_out = tc_out[_TC_N:]
    save_id_out = jnp.concatenate([save_id, idx], axis=-1)
    batch_indices = jnp.arange(_BEAM, dtype=jnp.int32) + (beam_size - _BEAM)
    max_logits_idx = idx[0]
    return (*tiled, idx, save_id_out, rp_out, prob, batch_indices, max_logits_idx)


# speedup vs baseline: 5.5713x; 1.0133x over previous
import jax
import jax.numpy as jnp
from jax import lax
from jax.experimental import pallas as pl
from jax.experimental.pallas import tpu as pltpu
from jax.experimental.pallas import tpu_sc as plsc

_BEAM = 4
_VOCAB = 100000
_PAD_V = 100096
_NEG = -1e30
_HEADS = 16
_SEQ = 2048
_HDIM = 64
_NKV = 12
_TC_N = 12
_SC_N = _NKV - _TC_N
_SC_ROWS = 16
_TC_CH = 4
_TC_NCH = _HEADS // _TC_CH


def _tc_body(*refs):
    kv_in = refs[:_TC_N]
    logits_ref, rp_ref, pen_ref = refs[_TC_N:_TC_N + 3]
    kv_out = refs[_TC_N + 3:2 * _TC_N + 3]
    idx_ref, prob_ref, rp_out_ref = refs[2 * _TC_N + 3:2 * _TC_N + 6]
    bufs = refs[2 * _TC_N + 6]
    rsem = refs[2 * _TC_N + 7]
    wsems = refs[2 * _TC_N + 8:2 * _TC_N + 12]

    total = _TC_N * _TC_NCH

    def read_for(t, slot):
        i, c = divmod(t, _TC_NCH)
        return pltpu.make_async_copy(
            kv_in[i].at[0, pl.ds(c * _TC_CH, _TC_CH)], bufs.at[slot], rsem.at[slot])

    def writes_for(t, slot):
        i, c = divmod(t, _TC_NCH)
        return [
            pltpu.make_async_copy(
                bufs.at[slot], kv_out[i].at[b, pl.ds(c * _TC_CH, _TC_CH)],
                wsems[b].at[slot])
            for b in range(_BEAM)
        ]

    rd = {0: read_for(0, 0), 1: read_for(1, 1)}
    rd[0].start()
    rd[1].start()

    x = logits_ref[...]
    m = jnp.max(x)
    lse = jnp.log(jnp.sum(jnp.exp(x - m))) + m
    cols = lax.broadcasted_iota(jnp.int32, (1, _PAD_V), 1)
    vals, idxs = [], []
    xc = x
    for _ in range(_BEAM):
        mk = jnp.max(xc)
        ik = jnp.min(jnp.where(xc == mk, cols, _PAD_V))
        vals.append(mk)
        idxs.append(ik)
        xc = jnp.where(cols == ik, _NEG, xc)
    for k in range(_BEAM):
        idx_ref[k, 0] = idxs[k]
        prob_ref[k, 0] = vals[k] - lse
    rcols = lax.broadcasted_iota(jnp.int32, (1, _VOCAB), 1)
    mask = (
        (rcols == idxs[0]) | (rcols == idxs[1]) | (rcols == idxs[2]) | (rcols == idxs[3])
    )
    p = pen_ref[0]
    rp_out_ref[...] = rp_ref[...] * jnp.where(mask, p, jnp.float32(1.0))

    pending = {}
    for t in range(total):
        slot = t % 2
        if t >= 2:
            for w in pending[t - 2]:
                w.wait()
            rd[t] = read_for(t, slot)
            rd[t].start()
        rd[t].wait()
        ws = writes_for(t, slot)
        for w in ws:
            w.start()
        pending[t] = ws
    for t in (total - 2, total - 1):
        for w in pending[t]:
            w.wait()


def _tc_call(kvs_tc, logits, rp, pen):
    logits_pad = jnp.pad(logits, ((0, 0), (0, _PAD_V - _VOCAB)), constant_values=_NEG)
    hbm = pl.BlockSpec(memory_space=pltpu.MemorySpace.HBM)
    vmem = pl.BlockSpec(memory_space=pltpu.MemorySpace.VMEM)
    smem = pl.BlockSpec(memory_space=pltpu.MemorySpace.SMEM)
    return pl.pallas_call(
        _tc_body,
        in_specs=[hbm] * _TC_N + [vmem, vmem, smem],
        out_specs=[hbm] * _TC_N + [smem, smem, vmem],
        out_shape=(
            [jax.ShapeDtypeStruct((_BEAM, _HEADS, _HDIM, _SEQ), jnp.float32)] * _TC_N
            + [
                jax.ShapeDtypeStruct((_BEAM, 1), jnp.int32),
                jax.ShapeDtypeStruct((_BEAM, 1), jnp.float32),
                jax.ShapeDtypeStruct((_BEAM, _VOCAB), jnp.float32),
            ]
        ),
        scratch_shapes=[
            pltpu.VMEM((2, _TC_CH, _HDIM, _SEQ), jnp.float32),
            pltpu.SemaphoreType.DMA((2,)),
        ] + [pltpu.SemaphoreType.DMA((2,))] * _BEAM,
    )(*kvs_tc, logits_pad, rp, pen)


def _sc_body(*refs):
    kv_in = refs[:_SC_N]
    kv_out = refs[_SC_N:2 * _SC_N]
    buf, rsem, wsem = refs[2 * _SC_N:]

    cid_core = lax.axis_index("c")
    sid = lax.axis_index("s")
    wid = sid * 2 + cid_core
    nchunk = _HDIM // _SC_ROWS

    def src_dst(t):
        i, r = divmod(t, 2)
        cid = wid * 2 + r
        h = lax.shift_right_logical(cid, 2)
        r0 = (cid & (nchunk - 1)) * _SC_ROWS
        src = kv_in[i].at[0, h, pl.ds(r0, _SC_ROWS)]
        dsts = [kv_out[i].at[b, h, pl.ds(r0, _SC_ROWS)] for b in range(_BEAM)]
        return src, dsts

    total = _SC_N * 2
    reads = {}
    pend = {}
    src0, _ = src_dst(0)
    reads[0] = pltpu.make_async_copy(src0, buf.at[0], rsem.at[0])
    reads[0].start()
    for t in range(total):
        slot = t % 2
        nslot = (t + 1) % 2
        if t >= 1:
            for w in pend[t - 1]:
                w.wait()
        if t + 1 < total:
            srcn, _ = src_dst(t + 1)
            reads[t + 1] = pltpu.make_async_copy(srcn, buf.at[nslot], rsem.at[nslot])
            reads[t + 1].start()
        reads[t].wait()
        _, dsts = src_dst(t)
        ws = [pltpu.make_async_copy(buf.at[slot], d, wsem.at[slot]) for d in dsts]
        for w in ws:
            w.start()
        pend[t] = ws
    for w in pend[total - 1]:
        w.wait()


def _sc_call(kvs_sc):
    f = pl.kernel(
        _sc_body,
        out_type=[
            jax.ShapeDtypeStruct((_BEAM, _HEADS, _HDIM, _SEQ), jnp.float32)
        ] * _SC_N,
        mesh=plsc.VectorSubcoreMesh(core_axis_name="c", subcore_axis_name="s"),
        scratch_types=[
            pltpu.VMEM((2, _SC_ROWS, _SEQ), jnp.float32),
            pltpu.SemaphoreType.DMA((2,)),
            pltpu.SemaphoreType.DMA((2,)),
        ],
    )
    return f(*kvs_sc)


def kernel(kv_0, kv_1, kv_2, kv_3, kv_4, kv_5, kv_6, kv_7, kv_8, kv_9, kv_10,
           kv_11, logits, save_id, repeat_penality, penality_value, beam_size):
    kvs = [kv_0, kv_1, kv_2, kv_3, kv_4, kv_5, kv_6, kv_7, kv_8, kv_9, kv_10, kv_11]
    kvts = [jnp.swapaxes(kv, 2, 3) for kv in kvs]
    tc_out = _tc_call(kvts[:_TC_N], logits, repeat_penality, penality_value)
    tiled = [jnp.swapaxes(o, 2, 3) for o in tc_out[:_TC_N]]
    idx, prob, rp_out = tc_out[_TC_N:]
    save_id_out = jnp.concatenate([save_id, idx], axis=-1)
    batch_indices = jnp.arange(_BEAM, dtype=jnp.int32) + (beam_size - _BEAM)
    max_logits_idx = idx[0]
    return (*tiled, idx, save_id_out, rp_out, prob, batch_indices, max_logits_idx)


# speedup vs baseline: 5.6196x; 1.0087x over previous
import jax
import jax.numpy as jnp
from jax import lax
from jax.experimental import pallas as pl
from jax.experimental.pallas import tpu as pltpu
from jax.experimental.pallas import tpu_sc as plsc

_BEAM = 4
_VOCAB = 100000
_PAD_V = 100096
_NEG = -1e30
_HEADS = 16
_SEQ = 2048
_HDIM = 64
_NKV = 12
_TC_N = 2
_SC_N = _NKV - _TC_N
_SC_ROWS = 16
_SC_SLOTS = 3
_TC_CH = 4
_TC_NCH = _HEADS // _TC_CH
_TC_SLOTS = 4


def _tc_body(*refs):
    kv_in = refs[:_TC_N]
    logits_ref, sid_ref, rp_ref, pen_ref = refs[_TC_N:_TC_N + 4]
    kv_out = refs[_TC_N + 4:2 * _TC_N + 4]
    idx_ref, sid_out_ref, prob_ref, maxi_ref, rp_out_ref = refs[2 * _TC_N + 4:2 * _TC_N + 9]
    bufs = refs[2 * _TC_N + 9]
    rsem = refs[2 * _TC_N + 10]
    wsems = refs[2 * _TC_N + 11:2 * _TC_N + 11 + _BEAM]

    total = _TC_N * _TC_NCH

    def read_for(t):
        i, c = divmod(t, _TC_NCH)
        return pltpu.make_async_copy(
            kv_in[i].at[0, pl.ds(c * _TC_CH, _TC_CH)],
            bufs.at[t % _TC_SLOTS], rsem.at[t % _TC_SLOTS])

    def writes_for(t):
        i, c = divmod(t, _TC_NCH)
        return [
            pltpu.make_async_copy(
                bufs.at[t % _TC_SLOTS], kv_out[i].at[b, pl.ds(c * _TC_CH, _TC_CH)],
                wsems[b].at[t % _TC_SLOTS])
            for b in range(_BEAM)
        ]

    def topk_compute():
        x = logits_ref[...]
        m = jnp.max(x)
        lse = jnp.log(jnp.sum(jnp.exp(x - m))) + m
        cols = lax.broadcasted_iota(jnp.int32, (1, _PAD_V), 1)
        vals, idxs = [], []
        xc = x
        for _ in range(_BEAM):
            mk = jnp.max(xc)
            ik = jnp.min(jnp.where(xc == mk, cols, _PAD_V))
            vals.append(mk)
            idxs.append(ik)
            xc = jnp.where(cols == ik, _NEG, xc)
        for k in range(_BEAM):
            idx_ref[k, 0] = idxs[k]
            prob_ref[k, 0] = vals[k] - lse
            sid_out_ref[k, 0] = sid_ref[k, 0]
            sid_out_ref[k, 1] = idxs[k]
        maxi_ref[0] = idxs[0]
        rcols = lax.broadcasted_iota(jnp.int32, (1, _VOCAB), 1)
        mask = (
            (rcols == idxs[0]) | (rcols == idxs[1])
            | (rcols == idxs[2]) | (rcols == idxs[3])
        )
        p = pen_ref[0]
        rp_out_ref[...] = rp_ref[...] * jnp.where(mask, p, jnp.float32(1.0))

    rd = {0: read_for(0), 1: read_for(1)}
    rd[0].start()
    rd[1].start()

    pending = {}
    for t in range(total):
        if t == 2:
            topk_compute()
        if t - 2 >= 0:
            for w in pending[t - 2]:
                w.wait()
        if t + 2 < total:
            rd[t + 2] = read_for(t + 2)
            rd[t + 2].start()
        rd[t].wait()
        ws = writes_for(t)
        for w in ws:
            w.start()
        pending[t] = ws
    for t in (total - 2, total - 1):
        for w in pending[t]:
            w.wait()


def _tc_call(kvs_tc, logits, save_id, rp, pen):
    logits_pad = jnp.pad(logits, ((0, 0), (0, _PAD_V - _VOCAB)), constant_values=_NEG)
    hbm = pl.BlockSpec(memory_space=pltpu.MemorySpace.HBM)
    vmem = pl.BlockSpec(memory_space=pltpu.MemorySpace.VMEM)
    smem = pl.BlockSpec(memory_space=pltpu.MemorySpace.SMEM)
    return pl.pallas_call(
        _tc_body,
        in_specs=[hbm] * _TC_N + [vmem, smem, vmem, smem],
        out_specs=[hbm] * _TC_N + [smem, smem, smem, smem, vmem],
        out_shape=(
            [jax.ShapeDtypeStruct((_BEAM, _HEADS, _HDIM, _SEQ), jnp.float32)] * _TC_N
            + [
                jax.ShapeDtypeStruct((_BEAM, 1), jnp.int32),
                jax.ShapeDtypeStruct((_BEAM, 2), jnp.int32),
                jax.ShapeDtypeStruct((_BEAM, 1), jnp.float32),
                jax.ShapeDtypeStruct((1,), jnp.int32),
                jax.ShapeDtypeStruct((_BEAM, _VOCAB), jnp.float32),
            ]
        ),
        scratch_shapes=[
            pltpu.VMEM((_TC_SLOTS, _TC_CH, _HDIM, _SEQ), jnp.float32),
            pltpu.SemaphoreType.DMA((_TC_SLOTS,)),
        ] + [pltpu.SemaphoreType.DMA((_TC_SLOTS,))] * _BEAM,
    )(*kvs_tc, logits_pad, save_id, rp, pen)


def _sc_body(*refs):
    kv_in = refs[:_SC_N]
    kv_out = refs[_SC_N:2 * _SC_N]
    buf, rsem, wsem = refs[2 * _SC_N:]

    cid_core = lax.axis_index("c")
    sid = lax.axis_index("s")
    wid = sid * 2 + cid_core
    nchunk = _HDIM // _SC_ROWS

    def src_dst(t):
        i, r = divmod(t, 2)
        cid = wid * 2 + r
        h = lax.shift_right_logical(cid, 2)
        r0 = (cid & (nchunk - 1)) * _SC_ROWS
        src = kv_in[i].at[0, h, pl.ds(r0, _SC_ROWS)]
        dsts = [kv_out[i].at[b, h, pl.ds(r0, _SC_ROWS)] for b in range(_BEAM)]
        return src, dsts

    total = _SC_N * 2
    reads = {}
    pend = {}
    src0, _ = src_dst(0)
    reads[0] = pltpu.make_async_copy(src0, buf.at[0], rsem.at[0])
    reads[0].start()
    for t in range(total):
        slot = t % _SC_SLOTS
        if t - 2 >= 0:
            for w in pend[t - 2]:
                w.wait()
        if t + 1 < total:
            nslot = (t + 1) % _SC_SLOTS
            srcn, _ = src_dst(t + 1)
            reads[t + 1] = pltpu.make_async_copy(srcn, buf.at[nslot], rsem.at[nslot])
            reads[t + 1].start()
        reads[t].wait()
        _, dsts = src_dst(t)
        ws = [pltpu.make_async_copy(buf.at[slot], d, wsem.at[slot]) for d in dsts]
        for w in ws:
            w.start()
        pend[t] = ws
    for t in (total - 2, total - 1):
        for w in pend[t]:
            w.wait()


def _sc_call(kvs_sc):
    f = pl.kernel(
        _sc_body,
        out_type=[
            jax.ShapeDtypeStruct((_BEAM, _HEADS, _HDIM, _SEQ), jnp.float32)
        ] * _SC_N,
        mesh=plsc.VectorSubcoreMesh(core_axis_name="c", subcore_axis_name="s"),
        scratch_types=[
            pltpu.VMEM((_SC_SLOTS, _SC_ROWS, _SEQ), jnp.float32),
            pltpu.SemaphoreType.DMA((_SC_SLOTS,)),
            pltpu.SemaphoreType.DMA((_SC_SLOTS,)),
        ],
    )
    return f(*kvs_sc)


def kernel(kv_0, kv_1, kv_2, kv_3, kv_4, kv_5, kv_6, kv_7, kv_8, kv_9, kv_10,
           kv_11, logits, save_id, repeat_penality, penality_value, beam_size):
    kvs = [kv_0, kv_1, kv_2, kv_3, kv_4, kv_5, kv_6, kv_7, kv_8, kv_9, kv_10, kv_11]
    kvts = [jnp.swapaxes(kv, 2, 3) for kv in kvs]
    sc_out = _sc_call(kvts[_TC_N:])
    tc_out = _tc_call(kvts[:_TC_N], logits, save_id, repeat_penality, penality_value)
    tiled = [jnp.swapaxes(o, 2, 3) for o in list(tc_out[:_TC_N]) + list(sc_out)]
    idx, save_id_out, prob, max_logits_idx, rp_out = tc_out[_TC_N:]
    batch_indices = jnp.arange(_BEAM, dtype=jnp.int32) + (beam_size - _BEAM)
    return (*tiled, idx, save_id_out, rp_out, prob, batch_indices, max_logits_idx)


# speedup vs baseline: 5.7225x; 1.0183x over previous
import jax
import jax.numpy as jnp
from jax import lax
from jax.experimental import pallas as pl
from jax.experimental.pallas import tpu as pltpu
from jax.experimental.pallas import tpu_sc as plsc

_BEAM = 4
_VOCAB = 100000
_PAD_V = 100096
_NEG = -1e30
_HEADS = 16
_SEQ = 2048
_HDIM = 64
_NKV = 12
_TC_N = 3
_SC_N = _NKV - _TC_N
_SC_ROWS = 16
_SC_SLOTS = 3
_TC_CH = 8
_TC_NCH = _HEADS // _TC_CH
_TC_SLOTS = 4


def _tc_body(*refs):
    kv_in = refs[:_TC_N]
    logits_ref, sid_ref, rp_ref, pen_ref = refs[_TC_N:_TC_N + 4]
    kv_out = refs[_TC_N + 4:2 * _TC_N + 4]
    idx_ref, sid_out_ref, prob_ref, maxi_ref, rp_out_ref = refs[2 * _TC_N + 4:2 * _TC_N + 9]
    bufs = refs[2 * _TC_N + 9]
    rsem = refs[2 * _TC_N + 10]
    wsems = refs[2 * _TC_N + 11:2 * _TC_N + 11 + _BEAM]

    total = _TC_N * _TC_NCH

    def read_for(t):
        i, c = divmod(t, _TC_NCH)
        return pltpu.make_async_copy(
            kv_in[i].at[0, pl.ds(c * _TC_CH, _TC_CH)],
            bufs.at[t % _TC_SLOTS], rsem.at[t % _TC_SLOTS])

    def writes_for(t):
        i, c = divmod(t, _TC_NCH)
        return [
            pltpu.make_async_copy(
                bufs.at[t % _TC_SLOTS], kv_out[i].at[b, pl.ds(c * _TC_CH, _TC_CH)],
                wsems[b].at[t % _TC_SLOTS])
            for b in range(_BEAM)
        ]

    def topk_compute():
        x = logits_ref[...]
        m = jnp.max(x)
        lse = jnp.log(jnp.sum(jnp.exp(x - m))) + m
        cols = lax.broadcasted_iota(jnp.int32, (1, _PAD_V), 1)
        vals, idxs = [], []
        xc = x
        for _ in range(_BEAM):
            mk = jnp.max(xc)
            ik = jnp.min(jnp.where(xc == mk, cols, _PAD_V))
            vals.append(mk)
            idxs.append(ik)
            xc = jnp.where(cols == ik, _NEG, xc)
        for k in range(_BEAM):
            idx_ref[k, 0] = idxs[k]
            prob_ref[k, 0] = vals[k] - lse
            sid_out_ref[k, 0] = sid_ref[k, 0]
            sid_out_ref[k, 1] = idxs[k]
        maxi_ref[0] = idxs[0]
        rcols = lax.broadcasted_iota(jnp.int32, (1, _VOCAB), 1)
        mask = (
            (rcols == idxs[0]) | (rcols == idxs[1])
            | (rcols == idxs[2]) | (rcols == idxs[3])
        )
        p = pen_ref[0]
        rp_out_ref[...] = rp_ref[...] * jnp.where(mask, p, jnp.float32(1.0))

    rd = {0: read_for(0), 1: read_for(1)}
    rd[0].start()
    rd[1].start()

    pending = {}
    for t in range(total):
        if t == 2:
            topk_compute()
        if t - 2 >= 0:
            for w in pending[t - 2]:
                w.wait()
        if t + 2 < total:
            rd[t + 2] = read_for(t + 2)
            rd[t + 2].start()
        rd[t].wait()
        ws = writes_for(t)
        for w in ws:
            w.start()
        pending[t] = ws
    for t in (total - 2, total - 1):
        for w in pending[t]:
            w.wait()


def _tc_call(kvs_tc, logits, save_id, rp, pen):
    logits_pad = jnp.pad(logits, ((0, 0), (0, _PAD_V - _VOCAB)), constant_values=_NEG)
    hbm = pl.BlockSpec(memory_space=pltpu.MemorySpace.HBM)
    vmem = pl.BlockSpec(memory_space=pltpu.MemorySpace.VMEM)
    smem = pl.BlockSpec(memory_space=pltpu.MemorySpace.SMEM)
    return pl.pallas_call(
        _tc_body,
        in_specs=[hbm] * _TC_N + [vmem, smem, vmem, smem],
        out_specs=[hbm] * _TC_N + [smem, smem, smem, smem, vmem],
        out_shape=(
            [jax.ShapeDtypeStruct((_BEAM, _HEADS, _HDIM, _SEQ), jnp.float32)] * _TC_N
            + [
                jax.ShapeDtypeStruct((_BEAM, 1), jnp.int32),
                jax.ShapeDtypeStruct((_BEAM, 2), jnp.int32),
                jax.ShapeDtypeStruct((_BEAM, 1), jnp.float32),
                jax.ShapeDtypeStruct((1,), jnp.int32),
                jax.ShapeDtypeStruct((_BEAM, _VOCAB), jnp.float32),
            ]
        ),
        scratch_shapes=[
            pltpu.VMEM((_TC_SLOTS, _TC_CH, _HDIM, _SEQ), jnp.float32),
            pltpu.SemaphoreType.DMA((_TC_SLOTS,)),
        ] + [pltpu.SemaphoreType.DMA((_TC_SLOTS,))] * _BEAM,
    )(*kvs_tc, logits_pad, save_id, rp, pen)


def _sc_body(*refs):
    kv_in = refs[:_SC_N]
    kv_out = refs[_SC_N:2 * _SC_N]
    buf, rsem, wsem = refs[2 * _SC_N:]

    cid_core = lax.axis_index("c")
    sid = lax.axis_index("s")
    wid = sid * 2 + cid_core
    nchunk = _HDIM // _SC_ROWS

    def src_dst(t):
        i, r = divmod(t, 2)
        cid = wid * 2 + r
        h = lax.shift_right_logical(cid, 2)
        r0 = (cid & (nchunk - 1)) * _SC_ROWS
        src = kv_in[i].at[0, h, pl.ds(r0, _SC_ROWS)]
        dsts = [kv_out[i].at[b, h, pl.ds(r0, _SC_ROWS)] for b in range(_BEAM)]
        return src, dsts

    total = _SC_N * 2
    reads = {}
    pend = {}
    src0, _ = src_dst(0)
    reads[0] = pltpu.make_async_copy(src0, buf.at[0], rsem.at[0])
    reads[0].start()
    for t in range(total):
        slot = t % _SC_SLOTS
        if t - 2 >= 0:
            for w in pend[t - 2]:
                w.wait()
        if t + 1 < total:
            nslot = (t + 1) % _SC_SLOTS
            srcn, _ = src_dst(t + 1)
            reads[t + 1] = pltpu.make_async_copy(srcn, buf.at[nslot], rsem.at[nslot])
            reads[t + 1].start()
        reads[t].wait()
        _, dsts = src_dst(t)
        ws = [pltpu.make_async_copy(buf.at[slot], d, wsem.at[slot]) for d in dsts]
        for w in ws:
            w.start()
        pend[t] = ws
    for t in (total - 2, total - 1):
        for w in pend[t]:
            w.wait()


def _sc_call(kvs_sc):
    f = pl.kernel(
        _sc_body,
        out_type=[
            jax.ShapeDtypeStruct((_BEAM, _HEADS, _HDIM, _SEQ), jnp.float32)
        ] * _SC_N,
        mesh=plsc.VectorSubcoreMesh(core_axis_name="c", subcore_axis_name="s"),
        scratch_types=[
            pltpu.VMEM((_SC_SLOTS, _SC_ROWS, _SEQ), jnp.float32),
            pltpu.SemaphoreType.DMA((_SC_SLOTS,)),
            pltpu.SemaphoreType.DMA((_SC_SLOTS,)),
        ],
    )
    return f(*kvs_sc)


def kernel(kv_0, kv_1, kv_2, kv_3, kv_4, kv_5, kv_6, kv_7, kv_8, kv_9, kv_10,
           kv_11, logits, save_id, repeat_penality, penality_value, beam_size):
    kvs = [kv_0, kv_1, kv_2, kv_3, kv_4, kv_5, kv_6, kv_7, kv_8, kv_9, kv_10, kv_11]
    kvts = [jnp.swapaxes(kv, 2, 3) for kv in kvs]
    sc_out = _sc_call(kvts[_TC_N:])
    tc_out = _tc_call(kvts[:_TC_N], logits, save_id, repeat_penality, penality_value)
    tiled = [jnp.swapaxes(o, 2, 3) for o in list(tc_out[:_TC_N]) + list(sc_out)]
    idx, save_id_out, prob, max_logits_idx, rp_out = tc_out[_TC_N:]
    batch_indices = jnp.arange(_BEAM, dtype=jnp.int32) + (beam_size - _BEAM)
    return (*tiled, idx, save_id_out, rp_out, prob, batch_indices, max_logits_idx)


# speedup vs baseline: 5.8135x; 1.0159x over previous
import jax
import jax.numpy as jnp
from jax import lax
from jax.experimental import pallas as pl
from jax.experimental.pallas import tpu as pltpu
from jax.experimental.pallas import tpu_sc as plsc

_BEAM = 4
_VOCAB = 100000
_PAD_V = 100096
_NEG = -1e30
_HEADS = 16
_SEQ = 2048
_HDIM = 64
_NKV = 12
_TC_N = 4
_SC_N = _NKV - _TC_N
_SC_ROWS = 16
_SC_SLOTS = 3
_TC_CH = 8
_TC_NCH = _HEADS // _TC_CH
_TC_SLOTS = 4


def _tc_body(*refs):
    kv_in = refs[:_TC_N]
    logits_ref, sid_ref, rp_ref, pen_ref = refs[_TC_N:_TC_N + 4]
    kv_out = refs[_TC_N + 4:2 * _TC_N + 4]
    idx_ref, sid_out_ref, prob_ref, maxi_ref, rp_out_ref = refs[2 * _TC_N + 4:2 * _TC_N + 9]
    bufs = refs[2 * _TC_N + 9]
    rsem = refs[2 * _TC_N + 10]
    wsems = refs[2 * _TC_N + 11:2 * _TC_N + 11 + _BEAM]

    total = _TC_N * _TC_NCH

    def read_for(t):
        i, c = divmod(t, _TC_NCH)
        return pltpu.make_async_copy(
            kv_in[i].at[0, pl.ds(c * _TC_CH, _TC_CH)],
            bufs.at[t % _TC_SLOTS], rsem.at[t % _TC_SLOTS])

    def writes_for(t):
        i, c = divmod(t, _TC_NCH)
        return [
            pltpu.make_async_copy(
                bufs.at[t % _TC_SLOTS], kv_out[i].at[b, pl.ds(c * _TC_CH, _TC_CH)],
                wsems[b].at[t % _TC_SLOTS])
            for b in range(_BEAM)
        ]

    def topk_compute():
        x = logits_ref[...]
        m = jnp.max(x)
        lse = jnp.log(jnp.sum(jnp.exp(x - m))) + m
        cols = lax.broadcasted_iota(jnp.int32, (1, _PAD_V), 1)
        vals, idxs = [], []
        xc = x
        for _ in range(_BEAM):
            mk = jnp.max(xc)
            ik = jnp.min(jnp.where(xc == mk, cols, _PAD_V))
            vals.append(mk)
            idxs.append(ik)
            xc = jnp.where(cols == ik, _NEG, xc)
        for k in range(_BEAM):
            idx_ref[k, 0] = idxs[k]
            prob_ref[k, 0] = vals[k] - lse
            sid_out_ref[k, 0] = sid_ref[k, 0]
            sid_out_ref[k, 1] = idxs[k]
        maxi_ref[0] = idxs[0]
        rcols = lax.broadcasted_iota(jnp.int32, (1, _VOCAB), 1)
        mask = (
            (rcols == idxs[0]) | (rcols == idxs[1])
            | (rcols == idxs[2]) | (rcols == idxs[3])
        )
        p = pen_ref[0]
        rp_out_ref[...] = rp_ref[...] * jnp.where(mask, p, jnp.float32(1.0))

    rd = {0: read_for(0), 1: read_for(1)}
    rd[0].start()
    rd[1].start()

    pending = {}
    for t in range(total):
        if t == 2:
            topk_compute()
        if t - 2 >= 0:
            for w in pending[t - 2]:
                w.wait()
        if t + 2 < total:
            rd[t + 2] = read_for(t + 2)
            rd[t + 2].start()
        rd[t].wait()
        ws = writes_for(t)
        for w in ws:
            w.start()
        pending[t] = ws
    for t in (total - 2, total - 1):
        for w in pending[t]:
            w.wait()


def _tc_call(kvs_tc, logits, save_id, rp, pen):
    logits_pad = jnp.pad(logits, ((0, 0), (0, _PAD_V - _VOCAB)), constant_values=_NEG)
    hbm = pl.BlockSpec(memory_space=pltpu.MemorySpace.HBM)
    vmem = pl.BlockSpec(memory_space=pltpu.MemorySpace.VMEM)
    smem = pl.BlockSpec(memory_space=pltpu.MemorySpace.SMEM)
    return pl.pallas_call(
        _tc_body,
        in_specs=[hbm] * _TC_N + [vmem, smem, vmem, smem],
        out_specs=[hbm] * _TC_N + [smem, smem, smem, smem, vmem],
        out_shape=(
            [jax.ShapeDtypeStruct((_BEAM, _HEADS, _HDIM, _SEQ), jnp.float32)] * _TC_N
            + [
                jax.ShapeDtypeStruct((_BEAM, 1), jnp.int32),
                jax.ShapeDtypeStruct((_BEAM, 2), jnp.int32),
                jax.ShapeDtypeStruct((_BEAM, 1), jnp.float32),
                jax.ShapeDtypeStruct((1,), jnp.int32),
                jax.ShapeDtypeStruct((_BEAM, _VOCAB), jnp.float32),
            ]
        ),
        scratch_shapes=[
            pltpu.VMEM((_TC_SLOTS, _TC_CH, _HDIM, _SEQ), jnp.float32),
            pltpu.SemaphoreType.DMA((_TC_SLOTS,)),
        ] + [pltpu.SemaphoreType.DMA((_TC_SLOTS,))] * _BEAM,
    )(*kvs_tc, logits_pad, save_id, rp, pen)


def _sc_body(*refs):
    kv_in = refs[:_SC_N]
    kv_out = refs[_SC_N:2 * _SC_N]
    buf, rsem, wsem = refs[2 * _SC_N:]

    cid_core = lax.axis_index("c")
    sid = lax.axis_index("s")
    wid = sid * 2 + cid_core
    nchunk = _HDIM // _SC_ROWS

    def src_dst(t):
        i, r = divmod(t, 2)
        cid = wid * 2 + r
        h = lax.shift_right_logical(cid, 2)
        r0 = (cid & (nchunk - 1)) * _SC_ROWS
        src = kv_in[i].at[0, h, pl.ds(r0, _SC_ROWS)]
        dsts = [kv_out[i].at[b, h, pl.ds(r0, _SC_ROWS)] for b in range(_BEAM)]
        return src, dsts

    total = _SC_N * 2
    reads = {}
    pend = {}
    src0, _ = src_dst(0)
    reads[0] = pltpu.make_async_copy(src0, buf.at[0], rsem.at[0])
    reads[0].start()
    for t in range(total):
        slot = t % _SC_SLOTS
        if t - 2 >= 0:
            for w in pend[t - 2]:
                w.wait()
        if t + 1 < total:
            nslot = (t + 1) % _SC_SLOTS
            srcn, _ = src_dst(t + 1)
            reads[t + 1] = pltpu.make_async_copy(srcn, buf.at[nslot], rsem.at[nslot])
            reads[t + 1].start()
        reads[t].wait()
        _, dsts = src_dst(t)
        ws = [pltpu.make_async_copy(buf.at[slot], d, wsem.at[slot]) for d in dsts]
        for w in ws:
            w.start()
        pend[t] = ws
    for t in (total - 2, total - 1):
        for w in pend[t]:
            w.wait()


def _sc_call(kvs_sc):
    f = pl.kernel(
        _sc_body,
        out_type=[
            jax.ShapeDtypeStruct((_BEAM, _HEADS, _HDIM, _SEQ), jnp.float32)
        ] * _SC_N,
        mesh=plsc.VectorSubcoreMesh(core_axis_name="c", subcore_axis_name="s"),
        scratch_types=[
            pltpu.VMEM((_SC_SLOTS, _SC_ROWS, _SEQ), jnp.float32),
            pltpu.SemaphoreType.DMA((_SC_SLOTS,)),
            pltpu.SemaphoreType.DMA((_SC_SLOTS,)),
        ],
    )
    return f(*kvs_sc)


def kernel(kv_0, kv_1, kv_2, kv_3, kv_4, kv_5, kv_6, kv_7, kv_8, kv_9, kv_10,
           kv_11, logits, save_id, repeat_penality, penality_value, beam_size):
    kvs = [kv_0, kv_1, kv_2, kv_3, kv_4, kv_5, kv_6, kv_7, kv_8, kv_9, kv_10, kv_11]
    kvts = [jnp.swapaxes(kv, 2, 3) for kv in kvs]
    sc_out = _sc_call(kvts[_TC_N:])
    tc_out = _tc_call(kvts[:_TC_N], logits, save_id, repeat_penality, penality_value)
    tiled = [jnp.swapaxes(o, 2, 3) for o in list(tc_out[:_TC_N]) + list(sc_out)]
    idx, save_id_out, prob, max_logits_idx, rp_out = tc_out[_TC_N:]
    batch_indices = jnp.arange(_BEAM, dtype=jnp.int32) + (beam_size - _BEAM)
    return (*tiled, idx, save_id_out, rp_out, prob, batch_indices, max_logits_idx)


# speedup vs baseline: 5.9107x; 1.0167x over previous
import jax
import jax.numpy as jnp
from jax import lax
from jax.experimental import pallas as pl
from jax.experimental.pallas import tpu as pltpu
from jax.experimental.pallas import tpu_sc as plsc

_BEAM = 4
_VOCAB = 100000
_PAD_V = 100096
_NEG = -1e30
_HEADS = 16
_SEQ = 2048
_HDIM = 64
_NKV = 12
_TC_N = 5
_SC_N = _NKV - _TC_N
_SC_ROWS = 16
_SC_SLOTS = 3
_TC_CH = 8
_TC_NCH = _HEADS // _TC_CH
_TC_SLOTS = 4


def _tc_body(*refs):
    kv_in = refs[:_TC_N]
    logits_ref, sid_ref, rp_ref, pen_ref = refs[_TC_N:_TC_N + 4]
    kv_out = refs[_TC_N + 4:2 * _TC_N + 4]
    idx_ref, sid_out_ref, prob_ref, maxi_ref, rp_out_ref = refs[2 * _TC_N + 4:2 * _TC_N + 9]
    bufs = refs[2 * _TC_N + 9]
    rsem = refs[2 * _TC_N + 10]
    wsems = refs[2 * _TC_N + 11:2 * _TC_N + 11 + _BEAM]

    total = _TC_N * _TC_NCH

    def read_for(t):
        i, c = divmod(t, _TC_NCH)
        return pltpu.make_async_copy(
            kv_in[i].at[0, pl.ds(c * _TC_CH, _TC_CH)],
            bufs.at[t % _TC_SLOTS], rsem.at[t % _TC_SLOTS])

    def writes_for(t):
        i, c = divmod(t, _TC_NCH)
        return [
            pltpu.make_async_copy(
                bufs.at[t % _TC_SLOTS], kv_out[i].at[b, pl.ds(c * _TC_CH, _TC_CH)],
                wsems[b].at[t % _TC_SLOTS])
            for b in range(_BEAM)
        ]

    def topk_compute():
        x = logits_ref[...]
        m = jnp.max(x)
        lse = jnp.log(jnp.sum(jnp.exp(x - m))) + m
        cols = lax.broadcasted_iota(jnp.int32, (1, _PAD_V), 1)
        vals, idxs = [], []
        xc = x
        for _ in range(_BEAM):
            mk = jnp.max(xc)
            ik = jnp.min(jnp.where(xc == mk, cols, _PAD_V))
            vals.append(mk)
            idxs.append(ik)
            xc = jnp.where(cols == ik, _NEG, xc)
        for k in range(_BEAM):
            idx_ref[k, 0] = idxs[k]
            prob_ref[k, 0] = vals[k] - lse
            sid_out_ref[k, 0] = sid_ref[k, 0]
            sid_out_ref[k, 1] = idxs[k]
        maxi_ref[0] = idxs[0]
        rcols = lax.broadcasted_iota(jnp.int32, (1, _VOCAB), 1)
        mask = (
            (rcols == idxs[0]) | (rcols == idxs[1])
            | (rcols == idxs[2]) | (rcols == idxs[3])
        )
        p = pen_ref[0]
        rp_out_ref[...] = rp_ref[...] * jnp.where(mask, p, jnp.float32(1.0))

    rd = {0: read_for(0), 1: read_for(1)}
    rd[0].start()
    rd[1].start()

    pending = {}
    for t in range(total):
        if t == 2:
            topk_compute()
        if t - 2 >= 0:
            for w in pending[t - 2]:
                w.wait()
        if t + 2 < total:
            rd[t + 2] = read_for(t + 2)
            rd[t + 2].start()
        rd[t].wait()
        ws = writes_for(t)
        for w in ws:
            w.start()
        pending[t] = ws
    for t in (total - 2, total - 1):
        for w in pending[t]:
            w.wait()


def _tc_call(kvs_tc, logits, save_id, rp, pen):
    logits_pad = jnp.pad(logits, ((0, 0), (0, _PAD_V - _VOCAB)), constant_values=_NEG)
    hbm = pl.BlockSpec(memory_space=pltpu.MemorySpace.HBM)
    vmem = pl.BlockSpec(memory_space=pltpu.MemorySpace.VMEM)
    smem = pl.BlockSpec(memory_space=pltpu.MemorySpace.SMEM)
    return pl.pallas_call(
        _tc_body,
        in_specs=[hbm] * _TC_N + [vmem, smem, vmem, smem],
        out_specs=[hbm] * _TC_N + [smem, smem, smem, smem, vmem],
        out_shape=(
            [jax.ShapeDtypeStruct((_BEAM, _HEADS, _HDIM, _SEQ), jnp.float32)] * _TC_N
            + [
                jax.ShapeDtypeStruct((_BEAM, 1), jnp.int32),
                jax.ShapeDtypeStruct((_BEAM, 2), jnp.int32),
                jax.ShapeDtypeStruct((_BEAM, 1), jnp.float32),
                jax.ShapeDtypeStruct((1,), jnp.int32),
                jax.ShapeDtypeStruct((_BEAM, _VOCAB), jnp.float32),
            ]
        ),
        scratch_shapes=[
            pltpu.VMEM((_TC_SLOTS, _TC_CH, _HDIM, _SEQ), jnp.float32),
            pltpu.SemaphoreType.DMA((_TC_SLOTS,)),
        ] + [pltpu.SemaphoreType.DMA((_TC_SLOTS,))] * _BEAM,
    )(*kvs_tc, logits_pad, save_id, rp, pen)


def _sc_body(*refs):
    kv_in = refs[:_SC_N]
    kv_out = refs[_SC_N:2 * _SC_N]
    buf, rsem, wsem = refs[2 * _SC_N:]

    cid_core = lax.axis_index("c")
    sid = lax.axis_index("s")
    wid = sid * 2 + cid_core
    nchunk = _HDIM // _SC_ROWS

    def src_dst(t):
        i, r = divmod(t, 2)
        cid = wid * 2 + r
        h = lax.shift_right_logical(cid, 2)
        r0 = (cid & (nchunk - 1)) * _SC_ROWS
        src = kv_in[i].at[0, h, pl.ds(r0, _SC_ROWS)]
        dsts = [kv_out[i].at[b, h, pl.ds(r0, _SC_ROWS)] for b in range(_BEAM)]
        return src, dsts

    total = _SC_N * 2
    reads = {}
    pend = {}
    src0, _ = src_dst(0)
    reads[0] = pltpu.make_async_copy(src0, buf.at[0], rsem.at[0])
    reads[0].start()
    for t in range(total):
        slot = t % _SC_SLOTS
        if t - 2 >= 0:
            for w in pend[t - 2]:
                w.wait()
        if t + 1 < total:
            nslot = (t + 1) % _SC_SLOTS
            srcn, _ = src_dst(t + 1)
            reads[t + 1] = pltpu.make_async_copy(srcn, buf.at[nslot], rsem.at[nslot])
            reads[t + 1].start()
        reads[t].wait()
        _, dsts = src_dst(t)
        ws = [pltpu.make_async_copy(buf.at[slot], d, wsem.at[slot]) for d in dsts]
        for w in ws:
            w.start()
        pend[t] = ws
    for t in (total - 2, total - 1):
        for w in pend[t]:
            w.wait()


def _sc_call(kvs_sc):
    f = pl.kernel(
        _sc_body,
        out_type=[
            jax.ShapeDtypeStruct((_BEAM, _HEADS, _HDIM, _SEQ), jnp.float32)
        ] * _SC_N,
        mesh=plsc.VectorSubcoreMesh(core_axis_name="c", subcore_axis_name="s"),
        scratch_types=[
            pltpu.VMEM((_SC_SLOTS, _SC_ROWS, _SEQ), jnp.float32),
            pltpu.SemaphoreType.DMA((_SC_SLOTS,)),
            pltpu.SemaphoreType.DMA((_SC_SLOTS,)),
        ],
    )
    return f(*kvs_sc)


def kernel(kv_0, kv_1, kv_2, kv_3, kv_4, kv_5, kv_6, kv_7, kv_8, kv_9, kv_10,
           kv_11, logits, save_id, repeat_penality, penality_value, beam_size):
    kvs = [kv_0, kv_1, kv_2, kv_3, kv_4, kv_5, kv_6, kv_7, kv_8, kv_9, kv_10, kv_11]
    kvts = [jnp.swapaxes(kv, 2, 3) for kv in kvs]
    sc_out = _sc_call(kvts[_TC_N:])
    tc_out = _tc_call(kvts[:_TC_N], logits, save_id, repeat_penality, penality_value)
    tiled = [jnp.swapaxes(o, 2, 3) for o in list(tc_out[:_TC_N]) + list(sc_out)]
    idx, save_id_out, prob, max_logits_idx, rp_out = tc_out[_TC_N:]
    batch_indices = jnp.arange(_BEAM, dtype=jnp.int32) + (beam_size - _BEAM)
    return (*tiled, idx, save_id_out, rp_out, prob, batch_indices, max_logits_idx)
